# baseline (speedup 1.0000x reference)
import functools

import jax
import jax.numpy as jnp
from jax import lax
from jax.experimental import pallas as pl
from jax.experimental.pallas import tpu as pltpu
from jax.experimental.pallas import tpu_sc as plsc

N = 10000
E = 320000
HD = 128
N1 = N // 2
N2 = (3 * N) // 4
NSEG = N2 - N1
SEGP = 2560
NC = 2
NS = 16
NW = NC * NS
K = 128
EF = E // 2
CH = (EF + NW * K - 1) // (NW * K)
EP = NW * CH * K
SRCBITS = 13
SRCMASK = (1 << SRCBITS) - 1
RPT = SEGP // NS
NBUF = 4
TP1 = 5120


def _sc_segsum(table, enc, tp):
    mesh = plsc.VectorSubcoreMesh(core_axis_name="c", subcore_axis_name="s")

    @functools.partial(
        pl.kernel,
        out_type=jax.ShapeDtypeStruct((NC, SEGP, HD), jnp.float32),
        mesh=mesh,
        scratch_types=[
            pltpu.VMEM((CH, K), jnp.int32),
            pltpu.VMEM((CH, K), jnp.int32),
            pltpu.VMEM((CH, K), jnp.int32),
            [pltpu.VMEM((K, HD), jnp.float32)] * NBUF,
            pltpu.VMEM((RPT, HD), jnp.float32),
            pltpu.VMEM_SHARED((SEGP, HD), jnp.float32),
            [pltpu.SemaphoreType.DMA] * NBUF,
            [pltpu.SemaphoreType.DMA] * NBUF,
        ],
    )
    def k(table_hbm, enc_hbm, out_hbm, enc_v, src_v, dst_v, rows,
          stage_v, acc, gsem, ssem):
        c = lax.axis_index("c")
        s = lax.axis_index("s")
        wid = c * NS + s

        pltpu.sync_copy(enc_hbm.at[wid], enc_v)

        def zrow(i, _):
            def zcol(j, _):
                stage_v[i, pl.ds(j * 16, 16)] = jnp.zeros((16,), jnp.float32)
                return 0
            return lax.fori_loop(0, HD // 16, zcol, 0)
        lax.fori_loop(0, RPT, zrow, 0)
        pltpu.sync_copy(stage_v, acc.at[pl.ds(s * RPT, RPT)])

        def drow(j, _):
            def dcol(t, _):
                v = enc_v[j, pl.ds(t * 16, 16)]
                src_v[j, pl.ds(t * 16, 16)] = lax.bitwise_and(v, SRCMASK)
                dst_v[j, pl.ds(t * 16, 16)] = lax.shift_right_logical(v, SRCBITS)
                return 0
            return lax.fori_loop(0, K // 16, dcol, 0)
        lax.fori_loop(0, CH, drow, 0)
        plsc.subcore_barrier()

        def gstart(j, b):
            pltpu.async_copy(table_hbm.at[src_v.at[j]], rows[b], gsem[b])

        def gwait(b):
            pltpu.make_async_copy(table_hbm.at[src_v.at[0]], rows[b],
                                  gsem[b]).wait()

        def sstart(j, b):
            pltpu.async_copy(rows[b], acc.at[dst_v.at[j]], ssem[b], add=True)

        def swait(b):
            pltpu.make_async_copy(rows[b], acc.at[dst_v.at[0]],
                                  ssem[b]).wait()

        for b in range(NBUF):
            gstart(b, b)

        def body(i, _):
            j0 = i * NBUF
            for b in range(NBUF):
                gwait(b)
                sstart(j0 + b, b)
            for b in range(NBUF):
                jn = j0 + b + NBUF

                @pl.when(jn < CH)
                def _():
                    swait(b)
                    gstart(jn, b)
            return 0
        lax.fori_loop(0, CH // NBUF, body, 0)
        for b in range(NBUF):
            swait(b)

        plsc.subcore_barrier()
        pltpu.sync_copy(acc.at[pl.ds(s * RPT, RPT)], stage_v)
        pltpu.sync_copy(stage_v, out_hbm.at[c, pl.ds(s * RPT, RPT)])

    return k(table, enc)


def _mm(a, b):
    return jax.lax.dot_general(a, b, (((1,), (0,)), ((), ())),
                               preferred_element_type=jnp.float32)


def _res(x, w1, b1, w2, b2, w3, b3):
    h1 = jnp.tanh(_mm(x, w1) + b1)
    h2 = jnp.tanh(_mm(h1, w2) + b2)
    return _mm(h2 + x, w3) + b3


def _embed_kernel(x_ref, w_ref, b_ref, o_ref):
    o_ref[...] = jnp.tanh(_mm(x_ref[...], w_ref[...]) + b_ref[...])


def _tc_embed(x, w, b):
    return pl.pallas_call(
        _embed_kernel,
        out_shape=jax.ShapeDtypeStruct((N, HD), jnp.float32),
    )(x, w, b.reshape(1, HD))


def _front_kernel(m0, m1, e_ref, *refs):
    o_ref = refs[-1]
    w = [r[...] for r in refs[:-1]]
    (mpW1, mpB1, mpW2, mpB2, mpW3, mpB3,
     mp1W1, mp1B1, mp1W2, mp1B2, mp1W3, mp1B3,
     cW1a, cB1, cW2, cB2, cW3, cB3,
     c1W1, c1B1, c1W2, c1B2, c1W3, c1B3,
     nW1, nB1, nW2, nB2, nW3, nB3,
     n1W1, n1B1, n1W2, n1B2, n1W3, n1B3) = w

    msgs = m0[...] + m1[...]
    redux = jnp.tanh(_res(msgs, mpW1, mpB1, mpW2, mpB2, mpW3, mpB3))
    redux = jnp.tanh(_res(redux, mp1W1, mp1B1, mp1W2, mp1B2, mp1W3, mp1B3))

    e0 = e_ref[...]
    h1 = jnp.tanh(_mm(e0, cW1a) + cB1)
    h2 = jnp.tanh(_mm(h1, cW2) + cB2)
    ec = _mm(h2, cW3) + _mm(e0, cW3[0:HD, :]) + cB3
    e1 = jnp.tanh(ec)
    e1 = jnp.tanh(_res(e1, c1W1, c1B1, c1W2, c1B2, c1W3, c1B3))

    xc = jnp.concatenate([e1, redux], axis=-1)
    e2 = jnp.tanh(_res(xc, nW1, nB1, nW2, nB2, nW3, nB3))
    e2 = jnp.tanh(_res(e2, n1W1, n1B1, n1W2, n1B2, n1W3, n1B3))
    o_ref[...] = e2


def _tc_front(msgs2, e_pad, mp, mp1, comb, comb1, node, node1):
    def flat(p):
        (w1, b1), (w2, b2), (w3, b3) = p["l1"], p["l2"], p["l3"]
        return [w1, b1.reshape(1, -1), w2, b2.reshape(1, -1),
                w3, b3.reshape(1, -1)]
    cw = flat(comb)
    cw[0] = cw[0][0:HD, :]
    weights = flat(mp) + flat(mp1) + cw + flat(comb1) + flat(node) + flat(node1)
    return pl.pallas_call(
        _front_kernel,
        out_shape=jax.ShapeDtypeStruct((SEGP, HD), jnp.float32),
    )(msgs2[0], msgs2[1], e_pad, *weights)


def kernel(x, edge_index, params):
    half = E // 2
    src1 = (edge_index[0, :half] % N1).astype(jnp.int32)
    dst1 = (edge_index[1, :half] % NSEG).astype(jnp.int32)
    src2 = (edge_index[0, half:] % NSEG).astype(jnp.int32)
    dst2 = (edge_index[1, half:] % NSEG).astype(jnp.int32)

    pad_dst = (NSEG + jnp.arange(EP - EF, dtype=jnp.int32) % (SEGP - NSEG))

    def prep(src, dst):
        enc = jnp.concatenate([src, jnp.zeros((EP - EF,), jnp.int32)]) | (
            jnp.concatenate([dst, pad_dst]) << SRCBITS)
        return enc.reshape(NW, CH, K)

    enc1 = prep(src1, dst1)
    enc2 = prep(src2, dst2)

    we, be = params["embed"]
    embed_all = _tc_embed(x, we, be)

    e1 = jnp.pad(embed_all[N1:N2], ((0, SEGP - NSEG), (0, 0)))
    e2 = jnp.pad(embed_all[N2:], ((0, SEGP - (N - N2)), (0, 0)))

    p = params
    tab1 = jnp.pad(embed_all[:N1], ((0, TP1 - N1), (0, 0)))
    msgs1 = _sc_segsum(tab1, enc1, TP1)
    out1 = _tc_front(msgs1, e1, p["d2_mp"], p["d2_mp1"], p["d2_comb"],
                     p["d2_comb1"], p["d2_node"], p["d2_node1"])
    msgs2 = _sc_segsum(out1, enc2, SEGP)
    out2 = _tc_front(msgs2, e2, p["d3_mp"], p["d3_mp1"], p["d3_comb"],
                     p["d3_comb1"], p["d3_node"], p["d3_node1"])

    return jnp.concatenate(
        [embed_all[:N1], out1[:NSEG], out2[:NSEG]], axis=0)


# speedup vs baseline: 1.0602x; 1.0602x over previous
import functools

import jax
import jax.numpy as jnp
from jax import lax
from jax.experimental import pallas as pl
from jax.experimental.pallas import tpu as pltpu
from jax.experimental.pallas import tpu_sc as plsc

N = 10000
E = 320000
HD = 128
N1 = N // 2
N2 = (3 * N) // 4
NSEG = N2 - N1
SEGP = 2560
NC = 2
NS = 16
NW = NC * NS
K = 128
EF = E // 2
EP = 163840
CH0 = 64
CH1 = 16
TOTCH = NS * (CH0 + CH1)
CHPAD = NS * CH0 + NS * CH0
SRCBITS = 13
SRCMASK = (1 << SRCBITS) - 1
RPT = SEGP // NS
NBUF = 4
TP1 = 5120


def _sc_segsum(table, enc, tp):
    mesh = plsc.VectorSubcoreMesh(core_axis_name="c", subcore_axis_name="s")

    @functools.partial(
        pl.kernel,
        out_type=jax.ShapeDtypeStruct((NC, SEGP, HD), jnp.float32),
        mesh=mesh,
        scratch_types=[
            pltpu.VMEM((CH0, K), jnp.int32),
            pltpu.VMEM((CH0, K), jnp.int32),
            pltpu.VMEM((CH0, K), jnp.int32),
            [pltpu.VMEM((K, HD), jnp.float32)] * NBUF,
            pltpu.VMEM((RPT, HD), jnp.float32),
            pltpu.VMEM_SHARED((SEGP, HD), jnp.float32),
            [pltpu.SemaphoreType.DMA] * NBUF,
            [pltpu.SemaphoreType.DMA] * NBUF,
        ],
    )
    def k(table_hbm, enc_hbm, out_hbm, enc_v, src_v, dst_v, rows,
          stage_v, acc, gsem, ssem):
        c = lax.axis_index("c")
        s = lax.axis_index("s")
        chc = jnp.where(c == 0, CH0, CH1)
        base = jnp.where(c == 0, s * CH0, NS * CH0 + s * CH1)

        pltpu.sync_copy(enc_hbm.at[pl.ds(base, CH0)], enc_v)

        def zrow(i, _):
            def zcol(j, _):
                stage_v[i, pl.ds(j * 16, 16)] = jnp.zeros((16,), jnp.float32)
                return 0
            return lax.fori_loop(0, HD // 16, zcol, 0)
        lax.fori_loop(0, RPT, zrow, 0)
        pltpu.sync_copy(stage_v, acc.at[pl.ds(s * RPT, RPT)])

        def drow(j, _):
            def dcol(t, _):
                v = enc_v[j, pl.ds(t * 16, 16)]
                src_v[j, pl.ds(t * 16, 16)] = lax.bitwise_and(v, SRCMASK)
                dst_v[j, pl.ds(t * 16, 16)] = lax.shift_right_logical(v, SRCBITS)
                return 0
            return lax.fori_loop(0, K // 16, dcol, 0)
        lax.fori_loop(0, chc, drow, 0)
        plsc.subcore_barrier()

        def gstart(j, b):
            pltpu.async_copy(table_hbm.at[src_v.at[j]], rows[b], gsem[b])

        def gwait(b):
            pltpu.make_async_copy(table_hbm.at[src_v.at[0]], rows[b],
                                  gsem[b]).wait()

        def sstart(j, b):
            pltpu.async_copy(rows[b], acc.at[dst_v.at[j]], ssem[b], add=True)

        def swait(b):
            pltpu.make_async_copy(rows[b], acc.at[dst_v.at[0]],
                                  ssem[b]).wait()

        for b in range(NBUF):
            gstart(b, b)

        def body(i, _):
            j0 = i * NBUF
            for b in range(NBUF):
                gwait(b)
                sstart(j0 + b, b)
            for b in range(NBUF):
                jn = j0 + b + NBUF

                @pl.when(jn < chc)
                def _():
                    swait(b)
                    gstart(jn, b)
            return 0
        lax.fori_loop(0, chc // NBUF, body, 0)
        for b in range(NBUF):
            swait(b)

        plsc.subcore_barrier()
        pltpu.sync_copy(acc.at[pl.ds(s * RPT, RPT)], stage_v)
        pltpu.sync_copy(stage_v, out_hbm.at[c, pl.ds(s * RPT, RPT)])

    return k(table, enc)


def _mm(a, b):
    return jax.lax.dot_general(a, b, (((1,), (0,)), ((), ())),
                               preferred_element_type=jnp.float32)


def _res(x, w1, b1, w2, b2, w3, b3):
    h1 = jnp.tanh(_mm(x, w1) + b1)
    h2 = jnp.tanh(_mm(h1, w2) + b2)
    return _mm(h2 + x, w3) + b3


def _embed_kernel(x_ref, w_ref, b_ref, o_ref):
    o_ref[...] = jnp.tanh(_mm(x_ref[...], w_ref[...]) + b_ref[...])


def _tc_embed(x, w, b):
    return pl.pallas_call(
        _embed_kernel,
        out_shape=jax.ShapeDtypeStruct((N, HD), jnp.float32),
    )(x, w, b.reshape(1, HD))


def _front_kernel(m0, m1, e_ref, *refs):
    o_ref = refs[-1]
    w = [r[...] for r in refs[:-1]]
    (mpW1, mpB1, mpW2, mpB2, mpW3, mpB3,
     mp1W1, mp1B1, mp1W2, mp1B2, mp1W3, mp1B3,
     cW1a, cB1, cW2, cB2, cW3, cB3,
     c1W1, c1B1, c1W2, c1B2, c1W3, c1B3,
     nW1, nB1, nW2, nB2, nW3, nB3,
     n1W1, n1B1, n1W2, n1B2, n1W3, n1B3) = w

    msgs = m0[...] + m1[...]
    redux = jnp.tanh(_res(msgs, mpW1, mpB1, mpW2, mpB2, mpW3, mpB3))
    redux = jnp.tanh(_res(redux, mp1W1, mp1B1, mp1W2, mp1B2, mp1W3, mp1B3))

    e0 = e_ref[...]
    h1 = jnp.tanh(_mm(e0, cW1a) + cB1)
    h2 = jnp.tanh(_mm(h1, cW2) + cB2)
    ec = _mm(h2, cW3) + _mm(e0, cW3[0:HD, :]) + cB3
    e1 = jnp.tanh(ec)
    e1 = jnp.tanh(_res(e1, c1W1, c1B1, c1W2, c1B2, c1W3, c1B3))

    xc = jnp.concatenate([e1, redux], axis=-1)
    e2 = jnp.tanh(_res(xc, nW1, nB1, nW2, nB2, nW3, nB3))
    e2 = jnp.tanh(_res(e2, n1W1, n1B1, n1W2, n1B2, n1W3, n1B3))
    o_ref[...] = e2


def _tc_front(msgs2, e_pad, mp, mp1, comb, comb1, node, node1):
    def flat(p):
        (w1, b1), (w2, b2), (w3, b3) = p["l1"], p["l2"], p["l3"]
        return [w1, b1.reshape(1, -1), w2, b2.reshape(1, -1),
                w3, b3.reshape(1, -1)]
    cw = flat(comb)
    cw[0] = cw[0][0:HD, :]
    weights = flat(mp) + flat(mp1) + cw + flat(comb1) + flat(node) + flat(node1)
    return pl.pallas_call(
        _front_kernel,
        out_shape=jax.ShapeDtypeStruct((SEGP, HD), jnp.float32),
    )(msgs2[0], msgs2[1], e_pad, *weights)


def kernel(x, edge_index, params):
    half = E // 2
    src1 = (edge_index[0, :half] % N1).astype(jnp.int32)
    dst1 = (edge_index[1, :half] % NSEG).astype(jnp.int32)
    src2 = (edge_index[0, half:] % NSEG).astype(jnp.int32)
    dst2 = (edge_index[1, half:] % NSEG).astype(jnp.int32)

    pad_dst = (NSEG + jnp.arange(EP - EF, dtype=jnp.int32) % (SEGP - NSEG))

    def prep(src, dst):
        enc = jnp.concatenate([src, jnp.zeros((EP - EF,), jnp.int32)]) | (
            jnp.concatenate([dst, pad_dst]) << SRCBITS)
        enc = enc.reshape(TOTCH, K)
        return jnp.pad(enc, ((0, CHPAD - TOTCH), (0, 0)))

    enc1 = prep(src1, dst1)
    enc2 = prep(src2, dst2)

    we, be = params["embed"]
    embed_all = _tc_embed(x, we, be)

    e1 = jnp.pad(embed_all[N1:N2], ((0, SEGP - NSEG), (0, 0)))
    e2 = jnp.pad(embed_all[N2:], ((0, SEGP - (N - N2)), (0, 0)))

    p = params
    tab1 = jnp.pad(embed_all[:N1], ((0, TP1 - N1), (0, 0)))
    msgs1 = _sc_segsum(tab1, enc1, TP1)
    out1 = _tc_front(msgs1, e1, p["d2_mp"], p["d2_mp1"], p["d2_comb"],
                     p["d2_comb1"], p["d2_node"], p["d2_node1"])
    msgs2 = _sc_segsum(out1, enc2, SEGP)
    out2 = _tc_front(msgs2, e2, p["d3_mp"], p["d3_mp1"], p["d3_comb"],
                     p["d3_comb1"], p["d3_node"], p["d3_node1"])

    return jnp.concatenate(
        [embed_all[:N1], out1[:NSEG], out2[:NSEG]], axis=0)


# speedup vs baseline: 2.2647x; 2.1361x over previous
import functools

import jax
import jax.numpy as jnp
from jax import lax
from jax.experimental import pallas as pl
from jax.experimental.pallas import tpu as pltpu
from jax.experimental.pallas import tpu_sc as plsc

N = 10000
E = 320000
HD = 128
N1 = N // 2
N2 = (3 * N) // 4
NSEG = N2 - N1
SEGP = 2560
NC = 2
NS = 16
NW = NC * NS
K = 128
EF = E // 2
CH = (EF + NW * K - 1) // (NW * K)
EP = NW * CH * K
SRCBITS = 13
SRCMASK = (1 << SRCBITS) - 1
RPT = SEGP // NS
NBUF = 2
TP1 = 5120


def _sc_segsum(table, enc, tp):
    mesh = plsc.VectorSubcoreMesh(core_axis_name="c", subcore_axis_name="s")

    @functools.partial(
        pl.kernel,
        out_type=jax.ShapeDtypeStruct((NC, SEGP, HD), jnp.float32),
        mesh=mesh,
        scratch_types=[
            pltpu.VMEM((CH, K), jnp.int32),
            pltpu.VMEM((CH, K), jnp.int32),
            pltpu.VMEM((CH, K), jnp.int32),
            [pltpu.VMEM((K, HD), jnp.float32)] * NBUF,
            pltpu.VMEM((RPT, HD), jnp.float32),
            pltpu.VMEM_SHARED((tp, HD), jnp.float32),
            pltpu.VMEM_SHARED((SEGP, HD), jnp.float32),
            [pltpu.SemaphoreType.DMA] * NBUF,
            [pltpu.SemaphoreType.DMA] * NBUF,
        ],
    )
    def k(table_hbm, enc_hbm, out_hbm, enc_v, src_v, dst_v, rows,
          stage_v, tab, acc, gsem, ssem):
        c = lax.axis_index("c")
        s = lax.axis_index("s")
        wid = c * NS + s
        tpt = tp // NS

        pltpu.sync_copy(table_hbm.at[pl.ds(s * tpt, tpt)],
                        tab.at[pl.ds(s * tpt, tpt)])
        pltpu.sync_copy(enc_hbm.at[wid], enc_v)

        def zrow(i, _):
            def zcol(j, _):
                stage_v[i, pl.ds(j * 16, 16)] = jnp.zeros((16,), jnp.float32)
                return 0
            return lax.fori_loop(0, HD // 16, zcol, 0)
        lax.fori_loop(0, RPT, zrow, 0)
        pltpu.sync_copy(stage_v, acc.at[pl.ds(s * RPT, RPT)])

        def drow(j, _):
            def dcol(t, _):
                v = enc_v[j, pl.ds(t * 16, 16)]
                src_v[j, pl.ds(t * 16, 16)] = lax.bitwise_and(v, SRCMASK)
                dst_v[j, pl.ds(t * 16, 16)] = lax.shift_right_logical(v, SRCBITS)
                return 0
            return lax.fori_loop(0, K // 16, dcol, 0)
        lax.fori_loop(0, CH, drow, 0)
        plsc.subcore_barrier()

        def gstart(j, b):
            pltpu.async_copy(tab.at[src_v.at[j]], rows[b], gsem[b])

        def gwait(b):
            pltpu.make_async_copy(tab.at[src_v.at[0]], rows[b],
                                  gsem[b]).wait()

        def sstart(j, b):
            pltpu.async_copy(rows[b], acc.at[dst_v.at[j]], ssem[b], add=True)

        def swait(b):
            pltpu.make_async_copy(rows[b], acc.at[dst_v.at[0]],
                                  ssem[b]).wait()

        for b in range(NBUF):
            gstart(b, b)

        def body(i, _):
            j0 = i * NBUF
            for b in range(NBUF):
                gwait(b)
                sstart(j0 + b, b)
            for b in range(NBUF):
                jn = j0 + b + NBUF

                @pl.when(jn < CH)
                def _():
                    swait(b)
                    gstart(jn, b)
            return 0
        lax.fori_loop(0, CH // NBUF, body, 0)
        for b in range(NBUF):
            swait(b)

        plsc.subcore_barrier()
        pltpu.sync_copy(acc.at[pl.ds(s * RPT, RPT)], stage_v)
        pltpu.sync_copy(stage_v, out_hbm.at[c, pl.ds(s * RPT, RPT)])

    return k(table, enc)


def _mm(a, b):
    return jax.lax.dot_general(a, b, (((1,), (0,)), ((), ())),
                               preferred_element_type=jnp.float32)


def _res(x, w1, b1, w2, b2, w3, b3):
    h1 = jnp.tanh(_mm(x, w1) + b1)
    h2 = jnp.tanh(_mm(h1, w2) + b2)
    return _mm(h2 + x, w3) + b3


def _embed_kernel(x_ref, w_ref, b_ref, o_ref):
    o_ref[...] = jnp.tanh(_mm(x_ref[...], w_ref[...]) + b_ref[...])


def _tc_embed(x, w, b):
    return pl.pallas_call(
        _embed_kernel,
        out_shape=jax.ShapeDtypeStruct((N, HD), jnp.float32),
    )(x, w, b.reshape(1, HD))


def _front_kernel(m0, m1, e_ref, *refs):
    o_ref = refs[-1]
    w = [r[...] for r in refs[:-1]]
    (mpW1, mpB1, mpW2, mpB2, mpW3, mpB3,
     mp1W1, mp1B1, mp1W2, mp1B2, mp1W3, mp1B3,
     cW1a, cB1, cW2, cB2, cW3, cB3,
     c1W1, c1B1, c1W2, c1B2, c1W3, c1B3,
     nW1, nB1, nW2, nB2, nW3, nB3,
     n1W1, n1B1, n1W2, n1B2, n1W3, n1B3) = w

    msgs = m0[...] + m1[...]
    redux = jnp.tanh(_res(msgs, mpW1, mpB1, mpW2, mpB2, mpW3, mpB3))
    redux = jnp.tanh(_res(redux, mp1W1, mp1B1, mp1W2, mp1B2, mp1W3, mp1B3))

    e0 = e_ref[...]
    h1 = jnp.tanh(_mm(e0, cW1a) + cB1)
    h2 = jnp.tanh(_mm(h1, cW2) + cB2)
    ec = _mm(h2, cW3) + _mm(e0, cW3[0:HD, :]) + cB3
    e1 = jnp.tanh(ec)
    e1 = jnp.tanh(_res(e1, c1W1, c1B1, c1W2, c1B2, c1W3, c1B3))

    xc = jnp.concatenate([e1, redux], axis=-1)
    e2 = jnp.tanh(_res(xc, nW1, nB1, nW2, nB2, nW3, nB3))
    e2 = jnp.tanh(_res(e2, n1W1, n1B1, n1W2, n1B2, n1W3, n1B3))
    o_ref[...] = e2


def _tc_front(msgs2, e_pad, mp, mp1, comb, comb1, node, node1):
    def flat(p):
        (w1, b1), (w2, b2), (w3, b3) = p["l1"], p["l2"], p["l3"]
        return [w1, b1.reshape(1, -1), w2, b2.reshape(1, -1),
                w3, b3.reshape(1, -1)]
    cw = flat(comb)
    cw[0] = cw[0][0:HD, :]
    weights = flat(mp) + flat(mp1) + cw + flat(comb1) + flat(node) + flat(node1)
    return pl.pallas_call(
        _front_kernel,
        out_shape=jax.ShapeDtypeStruct((SEGP, HD), jnp.float32),
    )(msgs2[0], msgs2[1], e_pad, *weights)


def kernel(x, edge_index, params):
    half = E // 2
    src1 = (edge_index[0, :half] % N1).astype(jnp.int32)
    dst1 = (edge_index[1, :half] % NSEG).astype(jnp.int32)
    src2 = (edge_index[0, half:] % NSEG).astype(jnp.int32)
    dst2 = (edge_index[1, half:] % NSEG).astype(jnp.int32)

    pad_dst = (NSEG + jnp.arange(EP - EF, dtype=jnp.int32) % (SEGP - NSEG))

    def prep(src, dst):
        enc = jnp.concatenate([src, jnp.zeros((EP - EF,), jnp.int32)]) | (
            jnp.concatenate([dst, pad_dst]) << SRCBITS)
        return enc.reshape(NW, CH, K)

    enc1 = prep(src1, dst1)
    enc2 = prep(src2, dst2)

    we, be = params["embed"]
    embed_all = _tc_embed(x, we, be)

    e1 = jnp.pad(embed_all[N1:N2], ((0, SEGP - NSEG), (0, 0)))
    e2 = jnp.pad(embed_all[N2:], ((0, SEGP - (N - N2)), (0, 0)))

    p = params
    msgs1 = _sc_segsum(embed_all, enc1, TP1)
    out1 = _tc_front(msgs1, e1, p["d2_mp"], p["d2_mp1"], p["d2_comb"],
                     p["d2_comb1"], p["d2_node"], p["d2_node1"])
    msgs2 = _sc_segsum(out1, enc2, SEGP)
    out2 = _tc_front(msgs2, e2, p["d3_mp"], p["d3_mp1"], p["d3_comb"],
                     p["d3_comb1"], p["d3_node"], p["d3_node1"])

    return jnp.concatenate(
        [embed_all[:N1], out1[:NSEG], out2[:NSEG]], axis=0)
